# Initial kernel scaffold; baseline (speedup 1.0000x reference)
import jax
import jax.numpy as jnp
from jax.experimental import pallas as pl


def kernel(x, edge_index_1, edge_index_2, Wl1, bl1, Wr1, Wl2, bl2, Wr2):
    raise NotImplementedError("write your pallas kernel here")


# speedup vs baseline: 4.0876x; 4.0876x over previous
import functools

import jax
import jax.numpy as jnp
from jax import lax
from jax.experimental import pallas as pl
from jax.experimental.pallas import tpu as pltpu
from jax.experimental.pallas import tpu_sc as plsc

NC = 2
NS = 16
NW = NC * NS
LANES = 16
CHUNK = 128
D = 128


def _round_up(v, m):
    return (v + m - 1) // m * m


def _sc_aggregate(table, src, dst, n_out):
    e_pad = src.shape[0]
    per_w = e_pad // NW
    chunks = per_w // CHUNK
    n_pad = _round_up(n_out + 1, NS * 8)
    rpw = n_pad // NS

    mesh = plsc.VectorSubcoreMesh(core_axis_name="c", subcore_axis_name="s")

    @functools.partial(
        pl.kernel,
        mesh=mesh,
        out_type=(
            jax.ShapeDtypeStruct((NC, n_pad, D), jnp.float32),
            jax.ShapeDtypeStruct((NC, n_pad, D), jnp.float32),
        ),
        scratch_types=[
            pltpu.VMEM((CHUNK,), jnp.int32),
            pltpu.VMEM((CHUNK,), jnp.int32),
            pltpu.VMEM((CHUNK, D), jnp.float32),
            pltpu.VMEM((CHUNK, D), jnp.float32),
            pltpu.VMEM_SHARED((n_pad, D), jnp.float32),
            pltpu.VMEM_SHARED((n_pad, D), jnp.float32),
            pltpu.SemaphoreType.DMA,
        ],
    )
    def agg(table_hbm, src_hbm, dst_hbm, sum_out, cnt_out,
            src_v, dst_v, rows_v, ones_v, acc_s, cnt_s, sem):
        c = lax.axis_index("c")
        s = lax.axis_index("s")
        wid = s * NC + c
        zero16 = jnp.zeros((LANES,), jnp.float32)

        @pl.loop(0, CHUNK)
        def _(r):
            @pl.loop(0, D // LANES)
            def _(cc):
                rows_v[r, pl.ds(cc * LANES, LANES)] = zero16
                ones_v[r, pl.ds(cc * LANES, LANES)] = zero16

        base = s * rpw
        off = 0
        while off < rpw:
            m = min(CHUNK, rpw - off)
            pltpu.sync_copy(rows_v.at[pl.ds(0, m)], acc_s.at[pl.ds(base + off, m)])
            pltpu.sync_copy(ones_v.at[pl.ds(0, m)], cnt_s.at[pl.ds(base + off, m)])
            off += m

        one16 = jnp.ones((LANES,), jnp.float32)

        @pl.loop(0, CHUNK)
        def _(r):
            @pl.loop(0, D // LANES)
            def _(cc):
                ones_v[r, pl.ds(cc * LANES, LANES)] = one16

        plsc.subcore_barrier()

        ebase = wid * per_w

        @pl.loop(0, chunks)
        def _(i):
            b = ebase + i * CHUNK
            pltpu.sync_copy(src_hbm.at[pl.ds(b, CHUNK)], src_v)
            pltpu.sync_copy(dst_hbm.at[pl.ds(b, CHUNK)], dst_v)
            pltpu.async_copy(table_hbm.at[src_v], rows_v, sem).wait()
            pltpu.sync_copy(rows_v, acc_s.at[dst_v], add=True)
            pltpu.sync_copy(ones_v, cnt_s.at[dst_v], add=True)

        plsc.subcore_barrier()

        pltpu.sync_copy(acc_s.at[pl.ds(base, rpw)], sum_out.at[c, pl.ds(base, rpw)])
        pltpu.sync_copy(cnt_s.at[pl.ds(base, rpw)], cnt_out.at[c, pl.ds(base, rpw)])

    return agg(table, src, dst)


def _dense_body(sp_ref, cp_ref, xt_ref, wl_ref, bl_ref, wr_ref, o_ref):
    ssum = sp_ref[0] + sp_ref[1]
    cnt = cp_ref[0, :, 0:1] + cp_ref[1, :, 0:1]
    mean = ssum / jnp.maximum(cnt, 1.0)
    h = jnp.dot(mean, wl_ref[...], preferred_element_type=jnp.float32)
    h = h + bl_ref[...]
    h = h + jnp.dot(xt_ref[...], wr_ref[...], preferred_element_type=jnp.float32)
    o_ref[...] = jnp.maximum(h, 0.0)


def _dense1(sum_parts, cnt_parts, xt, WlT, bl, WrT):
    m = xt.shape[0]
    bm = 1000
    return pl.pallas_call(
        _dense_body,
        grid=(m // bm,),
        in_specs=[
            pl.BlockSpec((NC, bm, D), lambda i: (0, i, 0)),
            pl.BlockSpec((NC, bm, D), lambda i: (0, i, 0)),
            pl.BlockSpec((bm, D), lambda i: (i, 0)),
            pl.BlockSpec((D, D), lambda i: (0, 0)),
            pl.BlockSpec((1, D), lambda i: (0, 0)),
            pl.BlockSpec((D, D), lambda i: (0, 0)),
        ],
        out_specs=pl.BlockSpec((bm, D), lambda i: (i, 0)),
        out_shape=jax.ShapeDtypeStruct((m, D), jnp.float32),
    )(sum_parts[:, :m], cnt_parts[:, :m], xt, WlT, bl.reshape(1, D), WrT)


def _dense2_body(sp_ref, cp_ref, xt_ref, wl_ref, bl_ref, wr_ref, o_ref, ls_ref):
    ssum = sp_ref[0] + sp_ref[1]
    cnt = cp_ref[0, :, 0:1] + cp_ref[1, :, 0:1]
    mean = ssum / jnp.maximum(cnt, 1.0)
    o = jnp.dot(mean, wl_ref[...], preferred_element_type=jnp.float32)
    o = o + bl_ref[...]
    o = o + jnp.dot(xt_ref[...], wr_ref[...], preferred_element_type=jnp.float32)
    o_ref[...] = o
    mx = jnp.max(o, axis=-1, keepdims=True)
    e = jnp.exp(o - mx)
    lse = jnp.log(jnp.sum(e, axis=-1, keepdims=True)) + mx
    ls_ref[...] = o - lse


def _dense2(sum_parts, cnt_parts, xt, WlT, bl, WrT):
    m = xt.shape[0]
    return pl.pallas_call(
        _dense2_body,
        out_shape=(
            jax.ShapeDtypeStruct((m, D), jnp.float32),
            jax.ShapeDtypeStruct((m, D), jnp.float32),
        ),
    )(sum_parts[:, :m], cnt_parts[:, :m], xt, WlT, bl.reshape(1, D), WrT)


def _pad_edges(edge_index, n_out):
    src = edge_index[0].astype(jnp.int32)
    dst = edge_index[1].astype(jnp.int32)
    e = src.shape[0]
    e_pad = _round_up(e, NW * CHUNK)
    pad = e_pad - e
    if pad:
        src = jnp.concatenate([src, jnp.zeros((pad,), jnp.int32)])
        dst = jnp.concatenate([dst, jnp.full((pad,), n_out, jnp.int32)])
    return src, dst


def kernel(x, edge_index_1, edge_index_2, Wl1, bl1, Wr1, Wl2, bl2, Wr2):
    n1, n2 = 5000, 1000
    src1, dst1 = _pad_edges(edge_index_1, n1)
    src2, dst2 = _pad_edges(edge_index_2, n2)

    xt = x[:n1]
    s1, c1 = _sc_aggregate(xt, src1, dst1, n1)
    h = _dense1(s1, c1, xt, Wl1.T, bl1, Wr1.T)

    ht = h[:n2]
    s2, c2 = _sc_aggregate(ht, src2, dst2, n2)
    out, ls = _dense2(s2, c2, ht, Wl2.T, bl2, Wr2.T)
    return (out, ls)
